# Initial kernel scaffold; baseline (speedup 1.0000x reference)
import jax
import jax.numpy as jnp
from jax.experimental import pallas as pl


def kernel(tokens, params):
    raise NotImplementedError("write your pallas kernel here")


# speedup vs baseline: 8.4700x; 8.4700x over previous
import functools

import numpy as np
import jax
import jax.numpy as jnp
from jax.experimental import pallas as pl
from jax.experimental.pallas import tpu as pltpu

_B, _T, _C, _V = 2, 2048, 512, 32000
_N = 16
_H = _C // _N
_L = 32
_TP = 256
_TC = 128
_TR = 256
_VT = 3200
_G = 8

_F32 = jnp.float32


def _hmask():
    return jnp.asarray(np.kron(np.eye(_H, dtype=np.float32),
                               np.ones((_N, _N), np.float32)))


def _masks():
    tri = np.tril(np.ones((_L, _L), np.float32))
    strict = np.tril(np.ones((_L, _L), np.float32), -1)
    eye = np.eye(_L, dtype=np.float32)
    return jnp.asarray(np.stack([tri, strict, eye]))


def _dot(a, b, prec=None):
    return jnp.dot(a, b, preferred_element_type=_F32, precision=prec)


def _bdot(a, b, ca, cb, prec=None):
    return jax.lax.dot_general(
        a, b, (((ca,), (cb,)), ((0,), (0,))),
        preferred_element_type=_F32, precision=prec)


def _ln(x, g, b, eps=1e-5):
    m = jnp.mean(x, axis=-1, keepdims=True)
    c = x - m
    v = jnp.mean(c * c, axis=-1, keepdims=True)
    return c * jax.lax.rsqrt(v + eps) * g + b


def _embed_kernel(tok_ref, *refs):
    o_ref = refs[_G]
    for g in range(_G):
        o_ref[0, g, :] = refs[g][0, 0, :]


def _embed(tokens, emb):
    tok = tokens.reshape(-1)
    n = tok.shape[0] // _G
    emb3 = emb.reshape(_V, 1, _C)

    def _imap(g, i, tr):
        return (tr[i * _G + g], 0, 0)

    in_specs = [pl.BlockSpec((1, 1, _C), functools.partial(_imap, g))
                for g in range(_G)]
    out = pl.pallas_call(
        _embed_kernel,
        grid_spec=pltpu.PrefetchScalarGridSpec(
            num_scalar_prefetch=1,
            grid=(n,),
            in_specs=in_specs,
            out_specs=pl.BlockSpec((1, _G, _C), lambda i, tr: (i, 0, 0)),
        ),
        out_shape=jax.ShapeDtypeStruct((n, _G, _C), _F32),
        compiler_params=pltpu.CompilerParams(
            dimension_semantics=("arbitrary",)),
        name="embed_gather",
    )(tok, *([emb3] * _G))
    return out.reshape(_B, _T, _C)


def _pre_kernel(has_vmix, *refs):
    it = iter(refs)
    x_ref = next(it)
    vf_ref = next(it) if has_vmix else None
    mix_ref = next(it)
    vec_ref = next(it)
    w1_ref, w2_ref, a1_ref, a2_ref = next(it), next(it), next(it), next(it)
    if has_vmix:
        v1_ref, v2_ref = next(it), next(it)
    g1_ref, g2_ref = next(it), next(it)
    wr_ref, wk_ref, wv_ref, hm_ref = next(it), next(it), next(it), next(it)
    r_o, ew_o, k_o, v_o, a_o, b_o, g_o = (next(it) for _ in range(7))
    prev_ref = next(it)

    j = pl.program_id(1)
    x = x_ref[0]
    xln = _ln(x, vec_ref[4], vec_ref[5])

    @pl.when(j == 0)
    def _():
        prev_ref[...] = jnp.zeros_like(prev_ref)

    xs = jnp.concatenate([prev_ref[...], xln[:_TP - 1]], axis=0)
    prev_ref[...] = xln[_TP - 1:_TP]
    xx = xs - xln
    mix = mix_ref[...]
    xr = xln + xx * mix[0]
    xw = xln + xx * mix[1]
    xk = xln + xx * mix[2]
    xv = xln + xx * mix[3]
    xa = xln + xx * mix[4]
    xg = xln + xx * mix[5]

    r = _dot(xr, wr_ref[...])
    k = _dot(xk, wk_ref[...])
    v = _dot(xv, wv_ref[...])

    wraw = vec_ref[0] + _dot(jnp.tanh(_dot(xw, w1_ref[...])), w2_ref[...])
    nwr = -wraw
    sp = jnp.maximum(nwr, 0.0) + jnp.log1p(jnp.exp(-jnp.abs(nwr)))
    ew = jnp.exp(-sp - 0.5)

    aa = jax.nn.sigmoid(vec_ref[1] + _dot(_dot(xa, a1_ref[...]), a2_ref[...]))
    g = _dot(jax.nn.sigmoid(_dot(xg, g1_ref[...])), g2_ref[...])
    if has_vmix:
        lam = jax.nn.sigmoid(vec_ref[6]
                             + _dot(_dot(xv, v1_ref[...]), v2_ref[...]))
        v = v + (vf_ref[0] - v) * lam

    kk = k * vec_ref[2]
    ss = _dot(kk * kk, hm_ref[...])
    kkn = kk / jnp.maximum(jnp.sqrt(ss), 1e-12)
    kf = k * (1.0 + (aa - 1.0) * vec_ref[3])

    r_o[0] = r
    ew_o[0] = ew
    k_o[0] = kf
    v_o[0] = v
    a_o[0] = -kkn
    b_o[0] = kkn * aa
    g_o[0] = g


def _tmix_pre(x, tp, ln_g, ln_b, v_first):
    has_vmix = v_first is not None
    mixc = jnp.stack([tp['x_r'], tp['x_w'], tp['x_k'],
                      tp['x_v'], tp['x_a'], tp['x_g']])
    v0 = tp['v0'] if has_vmix else jnp.zeros((_C,), _F32)
    vecs = jnp.stack([tp['w0'], tp['a0'], tp['k_k'], tp['k_a'],
                      ln_g, ln_b, v0])

    act = pl.BlockSpec((1, _TP, _C), lambda bi, j: (bi, j, 0))
    full = lambda s: pl.BlockSpec(s, lambda bi, j: tuple([0] * len(s)))

    inputs = [x]
    in_specs = [act]
    if has_vmix:
        inputs.append(v_first)
        in_specs.append(act)
    inputs += [mixc, vecs, tp['w1'], tp['w2'], tp['a1'], tp['a2']]
    in_specs += [full((6, _C)), full((7, _C)), full((_C, 8)), full((8, _C)),
                 full((_C, 8)), full((8, _C))]
    if has_vmix:
        inputs += [tp['v1'], tp['v2']]
        in_specs += [full((_C, 8)), full((8, _C))]
    inputs += [tp['g1'], tp['g2'], tp['Wr'], tp['Wk'], tp['Wv'], _hmask()]
    in_specs += [full((_C, 8)), full((8, _C)), full((_C, _C)),
                 full((_C, _C)), full((_C, _C)), full((_C, _C))]

    sds = jax.ShapeDtypeStruct((_B, _T, _C), _F32)
    outs = pl.pallas_call(
        functools.partial(_pre_kernel, has_vmix),
        grid=(_B, _T // _TP),
        in_specs=in_specs,
        out_specs=[act] * 7,
        out_shape=[sds] * 7,
        scratch_shapes=[pltpu.VMEM((1, _C), _F32)],
        compiler_params=pltpu.CompilerParams(
            dimension_semantics=("parallel", "arbitrary")),
        name="tmix_pre",
    )(*inputs)
    return outs


def _scan_kernel(d_ref, m_ref, y_ref, s_ref):
    j = pl.program_id(1)

    @pl.when(j == 0)
    def _():
        s_ref[...] = jnp.zeros_like(s_ref)

    tri = jnp.broadcast_to(m_ref[0], (_H, _L, _L))
    strict = m_ref[1]
    incl = m_ref[0]
    eye = m_ref[2]

    for i in range(_TC // _L):
        sl = slice(i * _L, (i + 1) * _L)
        r = d_ref[0, 0, :, :, sl]
        ew = d_ref[1, 0, :, :, sl]
        k = d_ref[2, 0, :, :, sl]
        v = d_ref[3, 0, :, :, sl]
        a = d_ref[4, 0, :, :, sl]
        b = d_ref[5, 0, :, :, sl]

        sm = _bdot(ew, tri, 2, 2)
        e1 = jnp.exp(sm)
        qn = 1.0 / e1
        ah = a * qn * jnp.exp(ew)
        bh = b * e1
        kh = k * e1
        rh = r * qn

        ab = _bdot(ah, bh, 1, 1) * strict
        akm = _bdot(ah, kh, 1, 1) * strict
        rbm = _bdot(rh, bh, 1, 1) * incl
        rkm = _bdot(rh, kh, 1, 1) * incl

        tm = eye + ab
        p = ab
        for _ in range(4):
            p = _bdot(p, p, 2, 1)
            tm = tm + _bdot(tm, p, 2, 1)

        st = s_ref[...]
        ut = _bdot(st, ah, 1, 1) + _bdot(v, akm, 2, 2)
        cct = _bdot(ut, tm, 2, 2)
        yt = (_bdot(st, rh, 1, 1) + _bdot(cct, rbm, 2, 2)
              + _bdot(v, rkm, 2, 2))
        y_ref[0, :, :, sl] = yt

        qlc = qn[:, :, _L - 1:_L]
        s_ref[...] = (st + _bdot(bh, cct, 2, 2)
                      + _bdot(kh, v, 2, 2)) * qlc


def _wkv7(r, ew, k, v, a, b):
    st = jnp.stack([r, ew, k, v, a, b])
    st = st.reshape(6, _B, _T, _H, _N).transpose(0, 1, 3, 4, 2)
    y = pl.pallas_call(
        _scan_kernel,
        grid=(_B, _T // _TC),
        in_specs=[
            pl.BlockSpec((6, 1, _H, _N, _TC),
                         lambda bi, j: (0, bi, 0, 0, j)),
            pl.BlockSpec((3, _L, _L), lambda bi, j: (0, 0, 0)),
        ],
        out_specs=pl.BlockSpec((1, _H, _N, _TC),
                               lambda bi, j: (bi, 0, 0, j)),
        out_shape=jax.ShapeDtypeStruct((_B, _H, _N, _T), _F32),
        scratch_shapes=[pltpu.VMEM((_H, _N, _N), _F32)],
        compiler_params=pltpu.CompilerParams(
            dimension_semantics=("parallel", "arbitrary")),
        name="wkv7_scan",
    )(st, _masks())
    return y.transpose(0, 3, 1, 2).reshape(_B, _T, _C)


def _post_kernel(y_ref, r_ref, k_ref, v_ref, g_ref, x_ref,
                 vec_ref, wo_ref, hm_ref, o_ref):
    y = y_ref[0]
    hm = hm_ref[...]
    m = _dot(y, hm) * (1.0 / _N)
    c = y - m
    var = _dot(c * c, hm) * (1.0 / _N)
    gn = c * jax.lax.rsqrt(var + 0.00064) * vec_ref[0] + vec_ref[1]
    rk = _dot(r_ref[0] * k_ref[0] * vec_ref[2], hm)
    y2 = gn + rk * v_ref[0]
    o_ref[0] = x_ref[0] + _dot(y2 * g_ref[0], wo_ref[...])


def _tmix_post(y, r, k, v, g, x, tp):
    vecs = jnp.stack([tp['gn_g'], tp['gn_b'], tp['r_k'].reshape(_C)])
    act = pl.BlockSpec((1, _TP, _C), lambda bi, j: (bi, j, 0))
    full = lambda s: pl.BlockSpec(s, lambda bi, j: tuple([0] * len(s)))
    return pl.pallas_call(
        _post_kernel,
        grid=(_B, _T // _TP),
        in_specs=[act] * 6 + [full((3, _C)), full((_C, _C)), full((_C, _C))],
        out_specs=act,
        out_shape=jax.ShapeDtypeStruct((_B, _T, _C), _F32),
        compiler_params=pltpu.CompilerParams(
            dimension_semantics=("parallel", "arbitrary")),
        name="tmix_post",
    )(y, r, k, v, g, x, vecs, tp['Wo'], _hmask())


def _ffn_kernel(x_ref, vec_ref, wk_ref, wv_ref, o_ref, prev_ref):
    j = pl.program_id(1)
    x = x_ref[0]
    xln = _ln(x, vec_ref[0], vec_ref[1])

    @pl.when(j == 0)
    def _():
        prev_ref[...] = jnp.zeros_like(prev_ref)

    xs = jnp.concatenate([prev_ref[...], xln[:_TP - 1]], axis=0)
    prev_ref[...] = xln[_TP - 1:_TP]
    xk = xln + (xs - xln) * vec_ref[2]
    h = jnp.square(jnp.maximum(_dot(xk, wk_ref[...]), 0.0))
    o_ref[0] = x + _dot(h, wv_ref[...])


def _ffn(x, fp, ln_g, ln_b):
    vecs = jnp.stack([ln_g, ln_b, fp['x_k']])
    act = pl.BlockSpec((1, _TP, _C), lambda bi, j: (bi, j, 0))
    full = lambda s: pl.BlockSpec(s, lambda bi, j: tuple([0] * len(s)))
    return pl.pallas_call(
        _ffn_kernel,
        grid=(_B, _T // _TP),
        in_specs=[act, full((3, _C)), full((_C, 4 * _C)), full((4 * _C, _C))],
        out_specs=act,
        out_shape=jax.ShapeDtypeStruct((_B, _T, _C), _F32),
        scratch_shapes=[pltpu.VMEM((1, _C), _F32)],
        compiler_params=pltpu.CompilerParams(
            dimension_semantics=("parallel", "arbitrary")),
        name="ffn",
    )(x, vecs, fp['Wk'], fp['Wv'])


def _head_kernel(x_ref, lnw_ref, w_ref, b_ref, o_ref):
    xln = _ln(x_ref[...], lnw_ref[0], lnw_ref[1])
    o_ref[...] = _dot(xln, w_ref[...]) + b_ref[...]


def _head(x, ln_g, ln_b, wout, bout):
    x2 = x.reshape(_B * _T, _C)
    lnw = jnp.stack([ln_g, ln_b])
    out = pl.pallas_call(
        _head_kernel,
        grid=(_V // _VT, (_B * _T) // _TR),
        in_specs=[
            pl.BlockSpec((_TR, _C), lambda jv, i: (i, 0)),
            pl.BlockSpec((2, _C), lambda jv, i: (0, 0)),
            pl.BlockSpec((_C, _VT), lambda jv, i: (0, jv)),
            pl.BlockSpec((1, _VT), lambda jv, i: (0, jv)),
        ],
        out_specs=pl.BlockSpec((_TR, _VT), lambda jv, i: (i, jv)),
        out_shape=jax.ShapeDtypeStruct((_B * _T, _V), _F32),
        compiler_params=pltpu.CompilerParams(
            dimension_semantics=("parallel", "arbitrary")),
        name="head_proj",
    )(x2, lnw, wout, bout.reshape(1, _V))
    return out.reshape(_B, _T, _V)


def _layer(x, tp, fp, lna_g, lna_b, lnb_g, lnb_b, v_first):
    r, ew, k, v, a, b, g = _tmix_pre(x, tp, lna_g, lna_b, v_first)
    y = _wkv7(r, ew, k, v, a, b)
    x = _tmix_post(y, r, k, v, g, x, tp)
    x = _ffn(x, fp, lnb_g, lnb_b)
    return x, v


def kernel(tokens, params):
    p = params
    x = _embed(tokens, p['emb'])
    x, v_first = _layer(x, p['rwkv1'], p['ffn1'], p['ln1a_g'], p['ln1a_b'],
                        p['ln1b_g'], p['ln1b_b'], None)
    x, _ = _layer(x, p['rwkv2'], p['ffn2'], p['ln2a_g'], p['ln2a_b'],
                  p['ln2b_g'], p['ln2b_b'], v_first)
    return _head(x, p['lno_g'], p['lno_b'], p['Wout'], p['bout'])
